# baseline (speedup 1.0000x reference)
import functools

import jax
import jax.numpy as jnp
from jax import lax
from jax.experimental import pallas as pl
from jax.experimental.pallas import tpu as pltpu
from jax.experimental.pallas import tpu_sc as plsc

N = 4096
E = 65536
H = 2
F_IN = 128
F_OUT = 64
ALPHA = 0.2
PADF = 128
NC = 2
NS = 16
NW = NC * NS
ECHUNK = E // NW
NEG = -1e30


def _prep_body(x_ref, w_ref, a_ref, hpad_ref, s12_ref):
    h = jnp.dot(x_ref[...], w_ref[0], preferred_element_type=jnp.float32)
    hpad_ref[0, :, 0:F_OUT] = h
    hpad_ref[0, :, F_OUT:F_OUT + 1] = jnp.ones((N, 1), jnp.float32)
    hpad_ref[0, :, F_OUT + 1:PADF] = jnp.zeros((N, PADF - F_OUT - 1), jnp.float32)
    a1 = a_ref[0, 0:1, 0:F_OUT]
    a2 = a_ref[0, 0:1, F_OUT:2 * F_OUT]
    s12_ref[0, :, 0:1] = jnp.sum(h * a1, axis=1, keepdims=True)
    s12_ref[0, :, 1:2] = jnp.sum(h * a2, axis=1, keepdims=True)


def _prep(x, weight, a_flat):
    return pl.pallas_call(
        _prep_body,
        grid=(H,),
        in_specs=[
            pl.BlockSpec((N, F_IN), lambda hh: (0, 0)),
            pl.BlockSpec((1, F_IN, F_OUT), lambda hh: (hh, 0, 0)),
            pl.BlockSpec((1, 1, 2 * F_OUT), lambda hh: (hh, 0, 0)),
        ],
        out_specs=[
            pl.BlockSpec((1, N, PADF), lambda hh: (hh, 0, 0)),
            pl.BlockSpec((1, N, 2), lambda hh: (hh, 0, 0)),
        ],
        out_shape=[
            jax.ShapeDtypeStruct((H, N, PADF), jnp.float32),
            jax.ShapeDtypeStruct((H, N, 2), jnp.float32),
        ],
    )(x, weight, a_flat)


def _rowmax_kernel(srcs, tgts, ea0s, s12):
    mesh = plsc.VectorSubcoreMesh(core_axis_name="c", subcore_axis_name="s")

    @functools.partial(
        pl.kernel,
        mesh=mesh,
        compiler_params=pltpu.CompilerParams(needs_layout_passes=False),
        out_type=[
            jax.ShapeDtypeStruct((NW, H * N), jnp.float32),
            jax.ShapeDtypeStruct((H, E), jnp.float32),
        ],
        scratch_types=[
            pltpu.VMEM((ECHUNK,), jnp.int32),
            pltpu.VMEM((ECHUNK,), jnp.int32),
            pltpu.VMEM((ECHUNK,), jnp.float32),
            pltpu.VMEM((H * N * 2,), jnp.float32),
            pltpu.VMEM((H * N,), jnp.float32),
            pltpu.VMEM((H, ECHUNK), jnp.float32),
        ],
    )
    def body(srcs_hbm, tgts_hbm, ea_hbm, s12_hbm, pmax_hbm, lbuf_hbm,
             src_v, tgt_v, ea_v, s12_v, rm_v, l_v):
        wid = lax.axis_index("c") * NS + lax.axis_index("s")
        base = wid * ECHUNK
        pltpu.sync_copy(srcs_hbm.at[pl.ds(base, ECHUNK)], src_v)
        pltpu.sync_copy(tgts_hbm.at[pl.ds(base, ECHUNK)], tgt_v)
        pltpu.sync_copy(ea_hbm.at[pl.ds(base, ECHUNK)], ea_v)
        pltpu.sync_copy(s12_hbm, s12_v)

        def init_body(j, _):
            rm_v[pl.ds(j * 16, 16)] = jnp.full((16,), NEG, jnp.float32)
            return 0
        lax.fori_loop(0, H * N // 16, init_body, 0)

        def chunk_body(j, _):
            off = j * 16
            s16 = src_v[pl.ds(off, 16)]
            t16 = tgt_v[pl.ds(off, 16)]
            ea16 = ea_v[pl.ds(off, 16)]
            for hh in range(H):
                s1v = plsc.load_gather(s12_v, [(s16 + hh * N) * 2])
                s2v = plsc.load_gather(s12_v, [(t16 + hh * N) * 2 + 1])
                u = s1v + s2v
                l = jnp.where(u > 0, u, ALPHA * u) * ea16
                l_v[hh, pl.ds(off, 16)] = l
                idx16 = s16 + hh * N

                def smax_round(_, lv=l, iv=idx16):
                    cur = plsc.load_gather(rm_v, [iv])
                    need = lv > cur
                    plsc.store_scatter(rm_v, [iv], jnp.maximum(cur, lv),
                                       mask=need)
                    chk = plsc.load_gather(rm_v, [iv])
                    return jnp.any(lv > chk)

                lax.while_loop(lambda p: p, smax_round, jnp.bool_(True))
            return 0
        lax.fori_loop(0, ECHUNK // 16, chunk_body, 0)

        pltpu.sync_copy(rm_v, pmax_hbm.at[wid])
        for hh in range(H):
            pltpu.sync_copy(l_v.at[hh], lbuf_hbm.at[hh, pl.ds(base, ECHUNK)])

    return body(srcs, tgts, ea0s, s12)


def _maxmerge_body(pmax_ref, rm_ref):
    rm_ref[...] = jnp.max(pmax_ref[...], axis=0, keepdims=True)


def _maxmerge(pmax):
    return pl.pallas_call(
        _maxmerge_body,
        out_shape=jax.ShapeDtypeStruct((1, H * N), jnp.float32),
    )(pmax)


BATCH = 64


def _aggregate_kernel(srcs, tgts, validf, lbuf, rowmax, hpad_flat):
    mesh = plsc.VectorSubcoreMesh(core_axis_name="c", subcore_axis_name="s")

    @functools.partial(
        pl.kernel,
        mesh=mesh,
        compiler_params=pltpu.CompilerParams(needs_layout_passes=False),
        out_type=jax.ShapeDtypeStruct((NC, H * N, PADF), jnp.float32),
        scratch_types=[
            pltpu.VMEM((ECHUNK,), jnp.int32),
            pltpu.VMEM((ECHUNK,), jnp.int32),
            pltpu.VMEM((ECHUNK,), jnp.float32),
            pltpu.VMEM((H * N,), jnp.float32),
            pltpu.VMEM((H * ECHUNK,), jnp.float32),
            pltpu.VMEM((BATCH,), jnp.float32),
            pltpu.VMEM((BATCH,), jnp.float32),
            pltpu.VMEM((BATCH,), jnp.int32),
            pltpu.VMEM((BATCH,), jnp.int32),
            pltpu.VMEM((BATCH,), jnp.int32),
            pltpu.VMEM((BATCH,), jnp.int32),
            pltpu.VMEM((BATCH, PADF), jnp.float32),
            pltpu.VMEM((BATCH, PADF), jnp.float32),
            pltpu.VMEM((16, PADF), jnp.float32),
            pltpu.VMEM_SHARED((H * N, PADF), jnp.float32),
            pltpu.SemaphoreType.DMA,
            pltpu.SemaphoreType.DMA,
        ],
    )
    def body(srcs_hbm, tgts_hbm, val_hbm, lbuf_hbm, rm_hbm, hpad_hbm, out_hbm,
             src_v, tgt_v, val_v, rm_v, l_v, w0_v, w1_v, g0_v, g1_v,
             s0_v, s1_v, rows0_v, rows1_v, zero_v, acc, sem0, sem1):
        cid = lax.axis_index("c")
        sid = lax.axis_index("s")
        wid = cid * NS + sid
        base = wid * ECHUNK
        pltpu.sync_copy(srcs_hbm.at[pl.ds(base, ECHUNK)], src_v)
        pltpu.sync_copy(tgts_hbm.at[pl.ds(base, ECHUNK)], tgt_v)
        pltpu.sync_copy(val_hbm.at[pl.ds(base, ECHUNK)], val_v)
        pltpu.sync_copy(rm_hbm.at[0], rm_v)
        for hh in range(H):
            pltpu.sync_copy(lbuf_hbm.at[hh, pl.ds(base, ECHUNK)],
                            l_v.at[pl.ds(hh * ECHUNK, ECHUNK)])

        for r in range(16):
            for q in range(PADF // 16):
                zero_v[r, pl.ds(q * 16, 16)] = jnp.zeros((16,), jnp.float32)
        for q in range(H * N // NS // 16):
            pltpu.sync_copy(zero_v, acc.at[pl.ds(sid * (H * N // NS) + q * 16, 16)])
        plsc.subcore_barrier()

        NBH = ECHUNK // BATCH
        NB = H * NBH

        def stage(b, w_b, g_b, s_b, rows_b, sem_b):
            hh = b // NBH
            eoff = (b % NBH) * BATCH
            for j in range(BATCH // 16):
                o16 = eoff + j * 16
                s16 = src_v[pl.ds(o16, 16)]
                t16 = tgt_v[pl.ds(o16, 16)]
                m16 = plsc.load_gather(rm_v, [s16 + hh * N])
                lv16 = l_v[pl.ds(hh * ECHUNK + o16, 16)]
                w16 = jnp.exp(lv16 - m16) * val_v[pl.ds(o16, 16)]
                w_b[pl.ds(j * 16, 16)] = w16
                g_b[pl.ds(j * 16, 16)] = t16 + hh * N
                s_b[pl.ds(j * 16, 16)] = s16 + hh * N
            pltpu.async_copy(hpad_hbm.at[g_b], rows_b, sem_b)

        def drain(g_b, rows_b, sem_b):
            pltpu.make_async_copy(hpad_hbm.at[g_b], rows_b, sem_b).wait()

        def consume(w_b, s_b, rows_b):
            for j in range(BATCH // 16):
                w16 = w_b[pl.ds(j * 16, 16)]
                for k in range(16):
                    e = j * 16 + k
                    wsc = w16[k]
                    for q in range(5):
                        sl = pl.ds(q * 16, 16)
                        rows_b[e, sl] = rows_b[e, sl] * wsc
            pltpu.sync_copy(rows_b, acc.at[s_b], add=True)

        stage(jnp.int32(0), w0_v, g0_v, s0_v, rows0_v, sem0)

        def pair_body(p, _):
            b1 = 2 * p + 1
            stage(b1, w1_v, g1_v, s1_v, rows1_v, sem1)
            drain(g0_v, rows0_v, sem0)
            consume(w0_v, s0_v, rows0_v)
            b2 = 2 * p + 2

            @pl.when(b2 < NB)
            def _():
                stage(b2, w0_v, g0_v, s0_v, rows0_v, sem0)
            drain(g1_v, rows1_v, sem1)
            consume(w1_v, s1_v, rows1_v)
            return 0
        lax.fori_loop(0, NB // 2, pair_body, 0)

        plsc.subcore_barrier()
        rows_per_sub = H * N // NS
        pltpu.sync_copy(acc.at[pl.ds(sid * rows_per_sub, rows_per_sub)],
                        out_hbm.at[cid, pl.ds(sid * rows_per_sub, rows_per_sub)])

    return body(srcs, tgts, validf, lbuf, rowmax, hpad_flat)


def _finalize_body(part_ref, hpad_ref, bias_ref, out_ref):
    for hh in range(H):
        sl = pl.ds(hh * N, N)
        acc = part_ref[0, sl] + part_ref[1, sl]
        z = acc[:, F_OUT:F_OUT + 1]
        hp = acc[:, 0:F_OUT]
        hmean = jnp.sum(hpad_ref[hh, :, 0:F_OUT], axis=0,
                        keepdims=True) * (1.0 / N)
        zsafe = jnp.maximum(z, 1e-30)
        out_ref[:, pl.ds(hh * F_OUT, F_OUT)] = (
            jnp.where(z > 0, hp / zsafe, hmean) + bias_ref[hh, 0])


def _finalize(partial, hpad, bias):
    return pl.pallas_call(
        _finalize_body,
        out_shape=jax.ShapeDtypeStruct((N, H * F_OUT), jnp.float32),
    )(partial, hpad, bias)


def kernel(x, edge_list, edge_attr, weight, a, bias):
    src = edge_list[0].astype(jnp.int32)
    tgt = edge_list[1].astype(jnp.int32)
    key = src * N + tgt
    ksort, order = lax.sort((key, jnp.arange(E, dtype=jnp.int32)), num_keys=1)
    keep = jnp.concatenate(
        [ksort[1:] != ksort[:-1], jnp.ones((1,), bool)]).astype(jnp.float32)
    valid = jnp.zeros((E,), jnp.float32).at[order].set(keep)

    a_flat = a[:, :, 0].reshape(H, 1, 2 * F_OUT)
    hpad, s12 = _prep(x, weight, a_flat)

    pmax, lbuf = _rowmax_kernel(src, tgt, edge_attr[:, 0], s12.reshape(-1))
    rowmax = _maxmerge(pmax)
    partial = _aggregate_kernel(src, tgt, valid, lbuf, rowmax,
                                hpad.reshape(H * N, PADF))
    return _finalize(partial, hpad, bias)


# speedup vs baseline: 2.0573x; 2.0573x over previous
import functools

import jax
import jax.numpy as jnp
from jax import lax
from jax.experimental import pallas as pl
from jax.experimental.pallas import tpu as pltpu
from jax.experimental.pallas import tpu_sc as plsc

N = 4096
E = 65536
H = 2
F_IN = 128
F_OUT = 64
ALPHA = 0.2
PADF = 128
NC = 2
NS = 16
NW = NC * NS
ECHUNK = E // NW
NEG = -1e30


def _prep_body(x_ref, w_ref, a_ref, hpad_ref, s12_ref):
    h = jnp.dot(x_ref[...], w_ref[0], preferred_element_type=jnp.float32)
    hpad_ref[0, :, 0:F_OUT] = h
    hpad_ref[0, :, F_OUT:F_OUT + 1] = jnp.ones((N, 1), jnp.float32)
    hpad_ref[0, :, F_OUT + 1:PADF] = jnp.zeros((N, PADF - F_OUT - 1), jnp.float32)
    a1 = a_ref[0, 0:1, 0:F_OUT]
    a2 = a_ref[0, 0:1, F_OUT:2 * F_OUT]
    s12_ref[0, :, 0:1] = jnp.sum(h * a1, axis=1, keepdims=True)
    s12_ref[0, :, 1:2] = jnp.sum(h * a2, axis=1, keepdims=True)


def _prep(x, weight, a_flat):
    return pl.pallas_call(
        _prep_body,
        grid=(H,),
        in_specs=[
            pl.BlockSpec((N, F_IN), lambda hh: (0, 0)),
            pl.BlockSpec((1, F_IN, F_OUT), lambda hh: (hh, 0, 0)),
            pl.BlockSpec((1, 1, 2 * F_OUT), lambda hh: (hh, 0, 0)),
        ],
        out_specs=[
            pl.BlockSpec((1, N, PADF), lambda hh: (hh, 0, 0)),
            pl.BlockSpec((1, N, 2), lambda hh: (hh, 0, 0)),
        ],
        out_shape=[
            jax.ShapeDtypeStruct((H, N, PADF), jnp.float32),
            jax.ShapeDtypeStruct((H, N, 2), jnp.float32),
        ],
    )(x, weight, a_flat)


def _rowmax_kernel(srcs, tgts, ea0s, s12):
    mesh = plsc.VectorSubcoreMesh(core_axis_name="c", subcore_axis_name="s")

    @functools.partial(
        pl.kernel,
        mesh=mesh,
        compiler_params=pltpu.CompilerParams(needs_layout_passes=False),
        out_type=[
            jax.ShapeDtypeStruct((NW, H * N), jnp.float32),
            jax.ShapeDtypeStruct((H, E), jnp.float32),
        ],
        scratch_types=[
            pltpu.VMEM((ECHUNK,), jnp.int32),
            pltpu.VMEM((ECHUNK,), jnp.int32),
            pltpu.VMEM((ECHUNK,), jnp.float32),
            pltpu.VMEM((H * N * 2,), jnp.float32),
            pltpu.VMEM((H * N,), jnp.float32),
            pltpu.VMEM((H, ECHUNK), jnp.float32),
        ],
    )
    def body(srcs_hbm, tgts_hbm, ea_hbm, s12_hbm, pmax_hbm, lbuf_hbm,
             src_v, tgt_v, ea_v, s12_v, rm_v, l_v):
        wid = lax.axis_index("c") * NS + lax.axis_index("s")
        base = wid * ECHUNK
        pltpu.sync_copy(srcs_hbm.at[pl.ds(base, ECHUNK)], src_v)
        pltpu.sync_copy(tgts_hbm.at[pl.ds(base, ECHUNK)], tgt_v)
        pltpu.sync_copy(ea_hbm.at[pl.ds(base, ECHUNK)], ea_v)
        pltpu.sync_copy(s12_hbm, s12_v)

        def init_body(j, _):
            rm_v[pl.ds(j * 16, 16)] = jnp.full((16,), NEG, jnp.float32)
            return 0
        lax.fori_loop(0, H * N // 16, init_body, 0)

        def chunk_body(j, _):
            off = j * 16
            s16 = src_v[pl.ds(off, 16)]
            t16 = tgt_v[pl.ds(off, 16)]
            ea16 = ea_v[pl.ds(off, 16)]
            for hh in range(H):
                s1v = plsc.load_gather(s12_v, [(s16 + hh * N) * 2])
                s2v = plsc.load_gather(s12_v, [(t16 + hh * N) * 2 + 1])
                u = s1v + s2v
                l = jnp.where(u > 0, u, ALPHA * u) * ea16
                l_v[hh, pl.ds(off, 16)] = l
                idx16 = s16 + hh * N

                def smax_round(_, lv=l, iv=idx16):
                    cur = plsc.load_gather(rm_v, [iv])
                    need = lv > cur
                    plsc.store_scatter(rm_v, [iv], jnp.maximum(cur, lv),
                                       mask=need)
                    chk = plsc.load_gather(rm_v, [iv])
                    return jnp.any(lv > chk)

                lax.while_loop(lambda p: p, smax_round, jnp.bool_(True))
            return 0
        lax.fori_loop(0, ECHUNK // 16, chunk_body, 0)

        pltpu.sync_copy(rm_v, pmax_hbm.at[wid])
        for hh in range(H):
            pltpu.sync_copy(l_v.at[hh], lbuf_hbm.at[hh, pl.ds(base, ECHUNK)])

    return body(srcs, tgts, ea0s, s12)


def _maxmerge_body(pmax_ref, rm_ref):
    rm_ref[...] = jnp.max(pmax_ref[...], axis=0, keepdims=True)


def _maxmerge(pmax):
    return pl.pallas_call(
        _maxmerge_body,
        out_shape=jax.ShapeDtypeStruct((1, H * N), jnp.float32),
    )(pmax)


BATCH = 64


def _aggregate_kernel(srcs, tgts, validf, lbuf, rowmax, hpad_flat):
    mesh = plsc.VectorSubcoreMesh(core_axis_name="c", subcore_axis_name="s")

    @functools.partial(
        pl.kernel,
        mesh=mesh,
        compiler_params=pltpu.CompilerParams(needs_layout_passes=False),
        out_type=jax.ShapeDtypeStruct((NC, H * N, PADF), jnp.float32),
        scratch_types=[
            pltpu.VMEM((ECHUNK,), jnp.int32),
            pltpu.VMEM((ECHUNK,), jnp.int32),
            pltpu.VMEM((ECHUNK,), jnp.float32),
            pltpu.VMEM((H * N,), jnp.float32),
            pltpu.VMEM((H * ECHUNK,), jnp.float32),
            pltpu.VMEM((BATCH,), jnp.float32),
            pltpu.VMEM((BATCH,), jnp.float32),
            pltpu.VMEM((BATCH,), jnp.int32),
            pltpu.VMEM((BATCH,), jnp.int32),
            pltpu.VMEM((BATCH,), jnp.int32),
            pltpu.VMEM((BATCH,), jnp.int32),
            pltpu.VMEM((BATCH, PADF), jnp.float32),
            pltpu.VMEM((BATCH, PADF), jnp.float32),
            pltpu.VMEM((16, PADF), jnp.float32),
            pltpu.VMEM_SHARED((H * N, PADF), jnp.float32),
            pltpu.SemaphoreType.DMA,
            pltpu.SemaphoreType.DMA,
        ],
    )
    def body(srcs_hbm, tgts_hbm, val_hbm, lbuf_hbm, rm_hbm, hpad_hbm, out_hbm,
             src_v, tgt_v, val_v, rm_v, l_v, w0_v, w1_v, g0_v, g1_v,
             s0_v, s1_v, rows0_v, rows1_v, zero_v, acc, sem0, sem1):
        cid = lax.axis_index("c")
        sid = lax.axis_index("s")
        wid = cid * NS + sid
        base = wid * ECHUNK
        pltpu.sync_copy(srcs_hbm.at[pl.ds(base, ECHUNK)], src_v)
        pltpu.sync_copy(tgts_hbm.at[pl.ds(base, ECHUNK)], tgt_v)
        pltpu.sync_copy(val_hbm.at[pl.ds(base, ECHUNK)], val_v)
        pltpu.sync_copy(rm_hbm.at[0], rm_v)
        for hh in range(H):
            pltpu.sync_copy(lbuf_hbm.at[hh, pl.ds(base, ECHUNK)],
                            l_v.at[pl.ds(hh * ECHUNK, ECHUNK)])

        for r in range(16):
            for q in range(PADF // 16):
                zero_v[r, pl.ds(q * 16, 16)] = jnp.zeros((16,), jnp.float32)
        for q in range(H * N // NS // 16):
            pltpu.sync_copy(zero_v, acc.at[pl.ds(sid * (H * N // NS) + q * 16, 16)])
        plsc.subcore_barrier()

        NBH = ECHUNK // BATCH
        NB = H * NBH

        def stage(b, w_b, g_b, s_b, rows_b, sem_b):
            hh = b // NBH
            eoff = (b % NBH) * BATCH
            for j in range(BATCH // 16):
                o16 = eoff + j * 16
                s16 = src_v[pl.ds(o16, 16)]
                t16 = tgt_v[pl.ds(o16, 16)]
                m16 = plsc.load_gather(rm_v, [s16 + hh * N])
                lv16 = l_v[pl.ds(hh * ECHUNK + o16, 16)]
                w16 = jnp.exp(lv16 - m16) * val_v[pl.ds(o16, 16)]
                w_b[pl.ds(j * 16, 16)] = w16
                g_b[pl.ds(j * 16, 16)] = t16 + hh * N
                s_b[pl.ds(j * 16, 16)] = s16 + hh * N
            pltpu.async_copy(hpad_hbm.at[g_b], rows_b, sem_b)

        def drain(g_b, rows_b, sem_b):
            pltpu.make_async_copy(hpad_hbm.at[g_b], rows_b, sem_b).wait()

        def consume(w_b, s_b, rows_b):
            for j in range(BATCH // 16):
                w16 = w_b[pl.ds(j * 16, 16)]
                for k in range(16):
                    e = j * 16 + k
                    wsc = w16[k]
                    for q in range(5):
                        sl = pl.ds(q * 16, 16)
                        rows_b[e, sl] = rows_b[e, sl] * wsc
            pltpu.sync_copy(rows_b, acc.at[s_b], add=True)

        stage(jnp.int32(0), w0_v, g0_v, s0_v, rows0_v, sem0)

        def pair_body(p, _):
            b1 = 2 * p + 1
            stage(b1, w1_v, g1_v, s1_v, rows1_v, sem1)
            drain(g0_v, rows0_v, sem0)
            consume(w0_v, s0_v, rows0_v)
            b2 = 2 * p + 2

            @pl.when(b2 < NB)
            def _():
                stage(b2, w0_v, g0_v, s0_v, rows0_v, sem0)
            drain(g1_v, rows1_v, sem1)
            consume(w1_v, s1_v, rows1_v)
            return 0
        lax.fori_loop(0, NB // 2, pair_body, 0)

        plsc.subcore_barrier()
        rows_per_sub = H * N // NS
        pltpu.sync_copy(acc.at[pl.ds(sid * rows_per_sub, rows_per_sub)],
                        out_hbm.at[cid, pl.ds(sid * rows_per_sub, rows_per_sub)])

    return body(srcs, tgts, validf, lbuf, rowmax, hpad_flat)


def _finalize_body(part_ref, hpad_ref, bias_ref, out_ref):
    for hh in range(H):
        sl = pl.ds(hh * N, N)
        acc = part_ref[0, sl] + part_ref[1, sl]
        z = acc[:, F_OUT:F_OUT + 1]
        hp = acc[:, 0:F_OUT]
        hmean = jnp.sum(hpad_ref[hh, :, 0:F_OUT], axis=0,
                        keepdims=True) * (1.0 / N)
        zsafe = jnp.maximum(z, 1e-30)
        out_ref[:, pl.ds(hh * F_OUT, F_OUT)] = (
            jnp.where(z > 0, hp / zsafe, hmean) + bias_ref[hh, 0])


def _finalize(partial, hpad, bias):
    return pl.pallas_call(
        _finalize_body,
        out_shape=jax.ShapeDtypeStruct((N, H * F_OUT), jnp.float32),
    )(partial, hpad, bias)


def kernel(x, edge_list, edge_attr, weight, a, bias):
    src = edge_list[0].astype(jnp.int32)
    tgt = edge_list[1].astype(jnp.int32)
    key = src * N + tgt
    ksort, order = lax.sort((key, jnp.arange(E, dtype=jnp.int32)), num_keys=1)
    srcs = ksort >> 12
    tgts = ksort & (N - 1)
    keep = jnp.concatenate(
        [ksort[1:] != ksort[:-1], jnp.ones((1,), bool)]).astype(jnp.float32)
    ea0s = jnp.take(edge_attr[:, 0], order)

    a_flat = a[:, :, 0].reshape(H, 1, 2 * F_OUT)
    hpad, s12 = _prep(x, weight, a_flat)

    pmax, lbuf = _rowmax_kernel(srcs, tgts, ea0s, s12.reshape(-1))
    rowmax = _maxmerge(pmax)
    partial = _aggregate_kernel(srcs, tgts, keep, lbuf, rowmax,
                                hpad.reshape(H * N, PADF))
    return _finalize(partial, hpad, bias)
